# baseline (speedup 1.0000x reference)
import functools

import jax
import jax.numpy as jnp
from jax.experimental import pallas as pl
from jax.experimental.pallas import tpu as pltpu

_TOP_R = 100
_TB = 2048


def _dist_kernel(test_f_ref, train_f_ref, out_ref, *, n_train, tb):
    b = pl.program_id(0)
    tbin = (test_f_ref[...] > 0).astype(jnp.float32)
    u = 1.0 - 2.0 * tbin
    c = jnp.sum(tbin, axis=1, keepdims=True)
    trbin = (train_f_ref[...] > 0).astype(jnp.float32)
    d = jax.lax.dot_general(u, trbin, (((1,), (1,)), ((), ())),
                            preferred_element_type=jnp.float32)
    dist = c + d
    idx = b * tb + jax.lax.broadcasted_iota(jnp.int32, (1, tb), 1)
    sd = jnp.where(idx >= n_train, jnp.int32(100),
                   dist.astype(jnp.int32) - 64)
    out_ref[...] = sd.astype(jnp.int8)


def _search_kernel(sd_ref, d_ref, r_ref, pos_ref, lcnt_ref, acc_ref,
                   *, n_blocks, n_iters, q, tb):
    it = pl.program_id(0)
    b = pl.program_id(1)

    @pl.when(jnp.logical_and(it == 0, b == 0))
    def _init():
        pos_ref[...] = jnp.zeros((q, 1), jnp.float32)
        lcnt_ref[...] = jnp.zeros((q, 1), jnp.float32)

    bit = (jnp.int32(256) >> (it + 1)).astype(jnp.float32)

    pos = pos_ref[...]
    thr = pos + (bit - 65.0)
    sd = sd_ref[...].astype(jnp.float32)
    part = jnp.sum((sd <= thr).astype(jnp.float32), axis=1, keepdims=True)
    cnt = jnp.where(b == 0, part, acc_ref[...] + part)
    acc_ref[...] = cnt

    is_last = jnp.where(b == n_blocks - 1, 1.0, 0.0)
    below = (cnt < float(_TOP_R)).astype(jnp.float32) * is_last
    pos = pos + bit * below
    pos_ref[...] = pos
    lcnt = jnp.where(below > 0.0, cnt, lcnt_ref[...])
    lcnt_ref[...] = lcnt
    d_ref[...] = pos
    r_ref[...] = float(_TOP_R) - lcnt


def _cumsum_lanes(x):
    width = x.shape[1]
    k = 1
    while k < width:
        shifted = jnp.concatenate(
            [jnp.zeros((x.shape[0], k), x.dtype), x[:, :-k]], axis=1)
        x = x + shifted
        k *= 2
    return x


_CHUNK = 128


def _count_kernel(sd_ref, d_ref, r_ref, ty_ref, qy_ref, out_ref,
                  seen_ref, acc_ref, ct_ref, *, n_blocks, q, tb):
    b = pl.program_id(0)
    nch = tb // _CHUNK

    @pl.when(b == 0)
    def _init():
        seen_ref[...] = jnp.zeros((q, 1), jnp.float32)
        acc_ref[...] = jnp.zeros((q, 1), jnp.float32)
        chunk_id = jax.lax.broadcasted_iota(jnp.int32, (tb, 32), 0) // _CHUNK
        col = jax.lax.broadcasted_iota(jnp.int32, (tb, 32), 1)
        ct_ref[...] = (chunk_id < col).astype(jnp.float32)

    sd = sd_ref[...].astype(jnp.float32)
    dthr = d_ref[...] - 64.0
    match = (ty_ref[0] == qy_ref[...]).astype(jnp.float32)
    lt = (sd < dthr).astype(jnp.float32)
    eq = (sd == dthr).astype(jnp.float32)
    eqm = eq * match
    acc_ref[...] += jnp.sum(lt * match, axis=1, keepdims=True)

    ct = ct_ref[...]
    p_eq = jnp.dot(eq, ct, preferred_element_type=jnp.float32)
    p_em = jnp.dot(eqm, ct, preferred_element_type=jnp.float32)
    ec = p_eq[:, 1:nch + 1] - p_eq[:, :nch]
    em = p_em[:, 1:nch + 1] - p_em[:, :nch]
    s = (r_ref[...] - seen_ref[...]) - p_eq[:, :nch]
    ft = (s >= ec).astype(jnp.float32)
    pc = jnp.logical_and(s > 0.0, s < ec).astype(jnp.float32)
    acc_ref[...] += jnp.sum(ft * em, axis=1, keepdims=True)

    eqx = jnp.zeros((q, _CHUNK), jnp.float32)
    emx = jnp.zeros((q, _CHUNK), jnp.float32)
    for c in range(nch):
        w = pc[:, c:c + 1]
        eqx = eqx + w * eq[:, c * _CHUNK:(c + 1) * _CHUNK]
        emx = emx + w * eqm[:, c * _CHUNK:(c + 1) * _CHUNK]
    s_star = jnp.sum(pc * s, axis=1, keepdims=True)
    pre = _cumsum_lanes(eqx)
    acc_ref[...] += jnp.sum(emx * (pre <= s_star).astype(jnp.float32),
                            axis=1, keepdims=True)
    seen_ref[...] += p_eq[:, nch:nch + 1]

    @pl.when(b == n_blocks - 1)
    def _fin():
        out_ref[...] = (jnp.sum(acc_ref[...], keepdims=True)
                        / (q * float(_TOP_R)))


def kernel(train_f, train_y, test_f, test_y):
    q, k = test_f.shape
    n = train_f.shape[0]
    tb = _TB if n >= _TB else max(128, ((n + 127) // 128) * 128)
    nb = (n + tb - 1) // tb
    npad = nb * tb
    n_iters = 8

    train_f_pad = jnp.pad(train_f, ((0, npad - n), (0, 0)))
    ty = jnp.pad(train_y.astype(jnp.float32), (0, npad - n))
    ty3 = ty.reshape(nb, 1, tb)
    qy = test_y.astype(jnp.float32).reshape(q, 1)

    sdist = pl.pallas_call(
        functools.partial(_dist_kernel, n_train=n, tb=tb),
        grid=(nb,),
        in_specs=[
            pl.BlockSpec((q, k), lambda b: (0, 0)),
            pl.BlockSpec((tb, k), lambda b: (b, 0)),
        ],
        out_specs=pl.BlockSpec((q, tb), lambda b: (0, b)),
        out_shape=jax.ShapeDtypeStruct((q, npad), jnp.int8),
    )(test_f, train_f_pad)

    dvals, rvals = pl.pallas_call(
        functools.partial(_search_kernel, n_blocks=nb, n_iters=n_iters,
                          q=q, tb=tb),
        grid=(n_iters, nb),
        in_specs=[pl.BlockSpec((q, tb), lambda it, b: (0, b))],
        out_specs=[
            pl.BlockSpec((q, 1), lambda it, b: (0, 0)),
            pl.BlockSpec((q, 1), lambda it, b: (0, 0)),
        ],
        out_shape=[
            jax.ShapeDtypeStruct((q, 1), jnp.float32),
            jax.ShapeDtypeStruct((q, 1), jnp.float32),
        ],
        scratch_shapes=[pltpu.VMEM((q, 1), jnp.float32) for _ in range(3)],
    )(sdist)

    out = pl.pallas_call(
        functools.partial(_count_kernel, n_blocks=nb, q=q, tb=tb),
        grid=(nb,),
        in_specs=[
            pl.BlockSpec((q, tb), lambda b: (0, b)),
            pl.BlockSpec((q, 1), lambda b: (0, 0)),
            pl.BlockSpec((q, 1), lambda b: (0, 0)),
            pl.BlockSpec((1, 1, tb), lambda b: (b, 0, 0)),
            pl.BlockSpec((q, 1), lambda b: (0, 0)),
        ],
        out_specs=pl.BlockSpec((1, 1), lambda b: (0, 0)),
        out_shape=jax.ShapeDtypeStruct((1, 1), jnp.float32),
        scratch_shapes=(
            [pltpu.VMEM((q, 1), jnp.float32) for _ in range(2)]
            + [pltpu.VMEM((tb, 32), jnp.float32)]),
    )(sdist, dvals, rvals, ty3, qy)

    return out[0, 0]


# speedup vs baseline: 1.0600x; 1.0600x over previous
import functools

import jax
import jax.numpy as jnp
from jax.experimental import pallas as pl
from jax.experimental.pallas import tpu as pltpu

_TOP_R = 100
_TB = 2048


def _dist_kernel(test_f_ref, train_f_ref, out_ref, cnt0_ref, acc_ref,
                 *, n_train, tb, n_blocks):
    b = pl.program_id(0)
    tbin = (test_f_ref[...] > 0).astype(jnp.float32)
    u = 1.0 - 2.0 * tbin
    c = jnp.sum(tbin, axis=1, keepdims=True)
    trbin = (train_f_ref[...] > 0).astype(jnp.float32)
    d = jax.lax.dot_general(u, trbin, (((1,), (1,)), ((), ())),
                            preferred_element_type=jnp.float32)
    dist = c + d
    idx = b * tb + jax.lax.broadcasted_iota(jnp.int32, (1, tb), 1)
    sd = jnp.where(idx >= n_train, jnp.int32(100),
                   dist.astype(jnp.int32) - 64)
    out_ref[...] = sd.astype(jnp.int8)
    part = jnp.sum((sd <= 63).astype(jnp.float32), axis=1, keepdims=True)
    acc_ref[...] = jnp.where(b == 0, part, acc_ref[...] + part)

    @pl.when(b == n_blocks - 1)
    def _fin():
        cnt0_ref[...] = acc_ref[...]


def _search_kernel(sd_ref, cnt0_ref, d_ref, r_ref, pos_ref, lcnt_ref,
                   acc_ref, *, n_blocks, n_iters, q, tb):
    it = pl.program_id(0)
    b = pl.program_id(1)

    @pl.when(jnp.logical_and(it == 0, b == 0))
    def _init():
        c0 = cnt0_ref[...]
        below0 = c0 < float(_TOP_R)
        pos_ref[...] = jnp.where(below0, 128.0, 0.0)
        lcnt_ref[...] = jnp.where(below0, c0, 0.0)

    bit = (jnp.int32(256) >> (it + 2)).astype(jnp.float32)

    thr = pos_ref[...] + (bit - 65.0)
    sd = sd_ref[...].astype(jnp.float32)
    part = jnp.sum((sd <= thr).astype(jnp.float32), axis=1, keepdims=True)
    cnt = jnp.where(b == 0, part, acc_ref[...] + part)
    acc_ref[...] = cnt

    @pl.when(b == n_blocks - 1)
    def _update():
        below = cnt < float(_TOP_R)
        pos = pos_ref[...] + jnp.where(below, bit, 0.0)
        pos_ref[...] = pos
        lcnt = jnp.where(below, cnt, lcnt_ref[...])
        lcnt_ref[...] = lcnt
        d_ref[...] = pos
        r_ref[...] = float(_TOP_R) - lcnt


def _cumsum_lanes(x):
    width = x.shape[1]
    k = 1
    while k < width:
        shifted = jnp.concatenate(
            [jnp.zeros((x.shape[0], k), x.dtype), x[:, :-k]], axis=1)
        x = x + shifted
        k *= 2
    return x


_CHUNK = 128


def _count_kernel(sd_ref, d_ref, r_ref, ty_ref, qy_ref, out_ref,
                  seen_ref, acc_ref, ct_ref, *, n_blocks, q, tb):
    b = pl.program_id(0)
    nch = tb // _CHUNK

    @pl.when(b == 0)
    def _init():
        seen_ref[...] = jnp.zeros((q, 1), jnp.float32)
        acc_ref[...] = jnp.zeros((q, 1), jnp.float32)
        chunk_id = jax.lax.broadcasted_iota(jnp.int32, (tb, 32), 0) // _CHUNK
        col = jax.lax.broadcasted_iota(jnp.int32, (tb, 32), 1)
        ct_ref[...] = (chunk_id < col).astype(jnp.float32)

    sd = sd_ref[...].astype(jnp.float32)
    dthr = d_ref[...] - 64.0
    match = (ty_ref[0] == qy_ref[...]).astype(jnp.float32)
    lt = (sd < dthr).astype(jnp.float32)
    eq = (sd == dthr).astype(jnp.float32)
    eqm = eq * match
    acc_ref[...] += jnp.sum(lt * match, axis=1, keepdims=True)

    ct = ct_ref[...]
    p_eq = jnp.dot(eq, ct, preferred_element_type=jnp.float32)
    p_em = jnp.dot(eqm, ct, preferred_element_type=jnp.float32)
    ec = p_eq[:, 1:nch + 1] - p_eq[:, :nch]
    em = p_em[:, 1:nch + 1] - p_em[:, :nch]
    s = (r_ref[...] - seen_ref[...]) - p_eq[:, :nch]
    ft = (s >= ec).astype(jnp.float32)
    pc = jnp.logical_and(s > 0.0, s < ec).astype(jnp.float32)
    acc_ref[...] += jnp.sum(ft * em, axis=1, keepdims=True)

    eqx = jnp.zeros((q, _CHUNK), jnp.float32)
    emx = jnp.zeros((q, _CHUNK), jnp.float32)
    for c in range(nch):
        w = pc[:, c:c + 1]
        eqx = eqx + w * eq[:, c * _CHUNK:(c + 1) * _CHUNK]
        emx = emx + w * eqm[:, c * _CHUNK:(c + 1) * _CHUNK]
    s_star = jnp.sum(pc * s, axis=1, keepdims=True)
    pre = _cumsum_lanes(eqx)
    acc_ref[...] += jnp.sum(emx * (pre <= s_star).astype(jnp.float32),
                            axis=1, keepdims=True)
    seen_ref[...] += p_eq[:, nch:nch + 1]

    @pl.when(b == n_blocks - 1)
    def _fin():
        out_ref[...] = (jnp.sum(acc_ref[...], keepdims=True)
                        / (q * float(_TOP_R)))


def kernel(train_f, train_y, test_f, test_y):
    q, k = test_f.shape
    n = train_f.shape[0]
    tb = _TB if n >= _TB else max(128, ((n + 127) // 128) * 128)
    nb = (n + tb - 1) // tb
    npad = nb * tb
    n_iters = 7

    train_f_pad = jnp.pad(train_f, ((0, npad - n), (0, 0)))
    ty = jnp.pad(train_y.astype(jnp.float32), (0, npad - n))
    ty3 = ty.reshape(nb, 1, tb)
    qy = test_y.astype(jnp.float32).reshape(q, 1)

    sdist, cnt0 = pl.pallas_call(
        functools.partial(_dist_kernel, n_train=n, tb=tb, n_blocks=nb),
        grid=(nb,),
        in_specs=[
            pl.BlockSpec((q, k), lambda b: (0, 0)),
            pl.BlockSpec((tb, k), lambda b: (b, 0)),
        ],
        out_specs=[
            pl.BlockSpec((q, tb), lambda b: (0, b)),
            pl.BlockSpec((q, 1), lambda b: (0, 0)),
        ],
        out_shape=[
            jax.ShapeDtypeStruct((q, npad), jnp.int8),
            jax.ShapeDtypeStruct((q, 1), jnp.float32),
        ],
        scratch_shapes=[pltpu.VMEM((q, 1), jnp.float32)],
    )(test_f, train_f_pad)

    dvals, rvals = pl.pallas_call(
        functools.partial(_search_kernel, n_blocks=nb, n_iters=n_iters,
                          q=q, tb=tb),
        grid=(n_iters, nb),
        in_specs=[pl.BlockSpec((q, tb), lambda it, b: (0, b)),
                  pl.BlockSpec((q, 1), lambda it, b: (0, 0))],
        out_specs=[
            pl.BlockSpec((q, 1), lambda it, b: (0, 0)),
            pl.BlockSpec((q, 1), lambda it, b: (0, 0)),
        ],
        out_shape=[
            jax.ShapeDtypeStruct((q, 1), jnp.float32),
            jax.ShapeDtypeStruct((q, 1), jnp.float32),
        ],
        scratch_shapes=[pltpu.VMEM((q, 1), jnp.float32) for _ in range(3)],
    )(sdist, cnt0)

    out = pl.pallas_call(
        functools.partial(_count_kernel, n_blocks=nb, q=q, tb=tb),
        grid=(nb,),
        in_specs=[
            pl.BlockSpec((q, tb), lambda b: (0, b)),
            pl.BlockSpec((q, 1), lambda b: (0, 0)),
            pl.BlockSpec((q, 1), lambda b: (0, 0)),
            pl.BlockSpec((1, 1, tb), lambda b: (b, 0, 0)),
            pl.BlockSpec((q, 1), lambda b: (0, 0)),
        ],
        out_specs=pl.BlockSpec((1, 1), lambda b: (0, 0)),
        out_shape=jax.ShapeDtypeStruct((1, 1), jnp.float32),
        scratch_shapes=(
            [pltpu.VMEM((q, 1), jnp.float32) for _ in range(2)]
            + [pltpu.VMEM((tb, 32), jnp.float32)]),
    )(sdist, dvals, rvals, ty3, qy)

    return out[0, 0]


# speedup vs baseline: 1.0608x; 1.0007x over previous
import functools

import jax
import jax.numpy as jnp
from jax.experimental import pallas as pl
from jax.experimental.pallas import tpu as pltpu

_TOP_R = 100
_TB = 2048


def _dist_kernel(test_f_ref, train_f_ref, out_ref, cnt0_ref, acc_ref,
                 *, n_train, tb, n_blocks):
    b = pl.program_id(0)
    tbin = (test_f_ref[...] > 0).astype(jnp.float32)
    u = (1.0 - 2.0 * tbin).astype(jnp.bfloat16)
    c = jnp.sum(tbin, axis=1, keepdims=True)
    trbin = (train_f_ref[...] > 0).astype(jnp.bfloat16)
    d = jax.lax.dot_general(u, trbin, (((1,), (1,)), ((), ())),
                            preferred_element_type=jnp.float32)
    dist = c + d
    idx = b * tb + jax.lax.broadcasted_iota(jnp.int32, (1, tb), 1)
    sd = jnp.where(idx >= n_train, jnp.int32(100),
                   dist.astype(jnp.int32) - 64)
    out_ref[...] = sd.astype(jnp.int8)
    part = jnp.sum((sd <= 63).astype(jnp.float32), axis=1, keepdims=True)
    acc_ref[...] = jnp.where(b == 0, part, acc_ref[...] + part)

    @pl.when(b == n_blocks - 1)
    def _fin():
        cnt0_ref[...] = acc_ref[...]


def _search_kernel(sd_ref, cnt0_ref, d_ref, r_ref, pos_ref, lcnt_ref,
                   acc_ref, *, n_blocks, n_iters, q, tb):
    it = pl.program_id(0)
    b = pl.program_id(1)

    @pl.when(jnp.logical_and(it == 0, b == 0))
    def _init():
        c0 = cnt0_ref[...]
        below0 = c0 < float(_TOP_R)
        pos_ref[...] = jnp.where(below0, 128.0, 0.0)
        lcnt_ref[...] = jnp.where(below0, c0, 0.0)

    bit = (jnp.int32(256) >> (it + 2)).astype(jnp.float32)

    thr = pos_ref[...] + (bit - 65.0)
    sd = sd_ref[...].astype(jnp.float32)
    part = jnp.sum((sd <= thr).astype(jnp.float32), axis=1, keepdims=True)
    cnt = jnp.where(b == 0, part, acc_ref[...] + part)
    acc_ref[...] = cnt

    @pl.when(b == n_blocks - 1)
    def _update():
        below = cnt < float(_TOP_R)
        pos = pos_ref[...] + jnp.where(below, bit, 0.0)
        pos_ref[...] = pos
        lcnt = jnp.where(below, cnt, lcnt_ref[...])
        lcnt_ref[...] = lcnt
        d_ref[...] = pos
        r_ref[...] = float(_TOP_R) - lcnt


def _cumsum_lanes(x):
    width = x.shape[1]
    k = 1
    while k < width:
        shifted = jnp.concatenate(
            [jnp.zeros((x.shape[0], k), x.dtype), x[:, :-k]], axis=1)
        x = x + shifted
        k *= 2
    return x


_CHUNK = 128


def _count_kernel(sd_ref, d_ref, r_ref, ty_ref, qy_ref, out_ref,
                  seen_ref, acc_ref, ct_ref, *, n_blocks, q, tb):
    b = pl.program_id(0)
    nch = tb // _CHUNK

    @pl.when(b == 0)
    def _init():
        seen_ref[...] = jnp.zeros((q, 1), jnp.float32)
        acc_ref[...] = jnp.zeros((q, 1), jnp.float32)
        chunk_id = jax.lax.broadcasted_iota(jnp.int32, (tb, 32), 0) // _CHUNK
        col = jax.lax.broadcasted_iota(jnp.int32, (tb, 32), 1)
        ct_ref[...] = (chunk_id < col).astype(jnp.float32)

    sd = sd_ref[...].astype(jnp.float32)
    dthr = d_ref[...] - 64.0
    match = (ty_ref[0] == qy_ref[...]).astype(jnp.float32)
    lt = (sd < dthr).astype(jnp.float32)
    eq = (sd == dthr).astype(jnp.float32)
    eqm = eq * match
    acc_ref[...] += jnp.sum(lt * match, axis=1, keepdims=True)

    ct = ct_ref[...]
    p_eq = jnp.dot(eq, ct, preferred_element_type=jnp.float32)
    p_em = jnp.dot(eqm, ct, preferred_element_type=jnp.float32)
    ec = p_eq[:, 1:nch + 1] - p_eq[:, :nch]
    em = p_em[:, 1:nch + 1] - p_em[:, :nch]
    s = (r_ref[...] - seen_ref[...]) - p_eq[:, :nch]
    ft = (s >= ec).astype(jnp.float32)
    pc = jnp.logical_and(s > 0.0, s < ec).astype(jnp.float32)
    acc_ref[...] += jnp.sum(ft * em, axis=1, keepdims=True)

    eqx = jnp.zeros((q, _CHUNK), jnp.float32)
    emx = jnp.zeros((q, _CHUNK), jnp.float32)
    for c in range(nch):
        w = pc[:, c:c + 1]
        eqx = eqx + w * eq[:, c * _CHUNK:(c + 1) * _CHUNK]
        emx = emx + w * eqm[:, c * _CHUNK:(c + 1) * _CHUNK]
    s_star = jnp.sum(pc * s, axis=1, keepdims=True)
    pre = _cumsum_lanes(eqx)
    acc_ref[...] += jnp.sum(emx * (pre <= s_star).astype(jnp.float32),
                            axis=1, keepdims=True)
    seen_ref[...] += p_eq[:, nch:nch + 1]

    @pl.when(b == n_blocks - 1)
    def _fin():
        out_ref[...] = (jnp.sum(acc_ref[...], keepdims=True)
                        / (q * float(_TOP_R)))


def kernel(train_f, train_y, test_f, test_y):
    q, k = test_f.shape
    n = train_f.shape[0]
    tb = _TB if n >= _TB else max(128, ((n + 127) // 128) * 128)
    nb = (n + tb - 1) // tb
    npad = nb * tb
    n_iters = 7

    train_f_pad = jnp.pad(train_f, ((0, npad - n), (0, 0)))
    ty = jnp.pad(train_y.astype(jnp.float32), (0, npad - n))
    ty3 = ty.reshape(nb, 1, tb)
    qy = test_y.astype(jnp.float32).reshape(q, 1)

    sdist, cnt0 = pl.pallas_call(
        functools.partial(_dist_kernel, n_train=n, tb=tb, n_blocks=nb),
        grid=(nb,),
        in_specs=[
            pl.BlockSpec((q, k), lambda b: (0, 0)),
            pl.BlockSpec((tb, k), lambda b: (b, 0)),
        ],
        out_specs=[
            pl.BlockSpec((q, tb), lambda b: (0, b)),
            pl.BlockSpec((q, 1), lambda b: (0, 0)),
        ],
        out_shape=[
            jax.ShapeDtypeStruct((q, npad), jnp.int8),
            jax.ShapeDtypeStruct((q, 1), jnp.float32),
        ],
        scratch_shapes=[pltpu.VMEM((q, 1), jnp.float32)],
    )(test_f, train_f_pad)

    dvals, rvals = pl.pallas_call(
        functools.partial(_search_kernel, n_blocks=nb, n_iters=n_iters,
                          q=q, tb=tb),
        grid=(n_iters, nb),
        in_specs=[pl.BlockSpec((q, tb), lambda it, b: (0, b)),
                  pl.BlockSpec((q, 1), lambda it, b: (0, 0))],
        out_specs=[
            pl.BlockSpec((q, 1), lambda it, b: (0, 0)),
            pl.BlockSpec((q, 1), lambda it, b: (0, 0)),
        ],
        out_shape=[
            jax.ShapeDtypeStruct((q, 1), jnp.float32),
            jax.ShapeDtypeStruct((q, 1), jnp.float32),
        ],
        scratch_shapes=[pltpu.VMEM((q, 1), jnp.float32) for _ in range(3)],
    )(sdist, cnt0)

    out = pl.pallas_call(
        functools.partial(_count_kernel, n_blocks=nb, q=q, tb=tb),
        grid=(nb,),
        in_specs=[
            pl.BlockSpec((q, tb), lambda b: (0, b)),
            pl.BlockSpec((q, 1), lambda b: (0, 0)),
            pl.BlockSpec((q, 1), lambda b: (0, 0)),
            pl.BlockSpec((1, 1, tb), lambda b: (b, 0, 0)),
            pl.BlockSpec((q, 1), lambda b: (0, 0)),
        ],
        out_specs=pl.BlockSpec((1, 1), lambda b: (0, 0)),
        out_shape=jax.ShapeDtypeStruct((1, 1), jnp.float32),
        scratch_shapes=(
            [pltpu.VMEM((q, 1), jnp.float32) for _ in range(2)]
            + [pltpu.VMEM((tb, 32), jnp.float32)]),
    )(sdist, dvals, rvals, ty3, qy)

    return out[0, 0]


# speedup vs baseline: 1.1607x; 1.0941x over previous
import functools

import jax
import jax.numpy as jnp
from jax.experimental import pallas as pl
from jax.experimental.pallas import tpu as pltpu

_TOP_R = 100
_TB = 4096


def _dist_kernel(test_f_ref, train_f_ref, out_ref, cnt0_ref, acc_ref,
                 *, n_train, tb, n_blocks):
    b = pl.program_id(0)
    tbin = (test_f_ref[...] > 0).astype(jnp.float32)
    u = (1.0 - 2.0 * tbin).astype(jnp.bfloat16)
    c = jnp.sum(tbin, axis=1, keepdims=True)
    trbin = (train_f_ref[...] > 0).astype(jnp.bfloat16)
    d = jax.lax.dot_general(u, trbin, (((1,), (1,)), ((), ())),
                            preferred_element_type=jnp.float32)
    dist = c + d
    idx = b * tb + jax.lax.broadcasted_iota(jnp.int32, (1, tb), 1)
    sd = jnp.where(idx >= n_train, jnp.int32(100),
                   dist.astype(jnp.int32) - 64)
    out_ref[...] = sd.astype(jnp.int8)
    part = jnp.sum((sd <= 63).astype(jnp.float32), axis=1, keepdims=True)
    acc_ref[...] = jnp.where(b == 0, part, acc_ref[...] + part)

    @pl.when(b == n_blocks - 1)
    def _fin():
        cnt0_ref[...] = acc_ref[...]


def _search_kernel(sd_ref, cnt0_ref, d_ref, r_ref, pos_ref, lcnt_ref,
                   acc_ref, *, n_blocks, n_iters, q, tb):
    it = pl.program_id(0)
    b = pl.program_id(1)

    @pl.when(jnp.logical_and(it == 0, b == 0))
    def _init():
        c0 = cnt0_ref[...]
        below0 = c0 < float(_TOP_R)
        pos_ref[...] = jnp.where(below0, 128.0, 0.0)
        lcnt_ref[...] = jnp.where(below0, c0, 0.0)

    bit = (jnp.int32(256) >> (it + 2)).astype(jnp.float32)

    thr = pos_ref[...] + (bit - 65.0)
    sd = sd_ref[...].astype(jnp.float32)
    part = jnp.sum((sd <= thr).astype(jnp.float32), axis=1, keepdims=True)
    cnt = jnp.where(b == 0, part, acc_ref[...] + part)
    acc_ref[...] = cnt

    @pl.when(b == n_blocks - 1)
    def _update():
        below = cnt < float(_TOP_R)
        pos = pos_ref[...] + jnp.where(below, bit, 0.0)
        pos_ref[...] = pos
        lcnt = jnp.where(below, cnt, lcnt_ref[...])
        lcnt_ref[...] = lcnt
        d_ref[...] = pos
        r_ref[...] = float(_TOP_R) - lcnt


def _cumsum_lanes(x):
    width = x.shape[1]
    k = 1
    while k < width:
        shifted = jnp.concatenate(
            [jnp.zeros((x.shape[0], k), x.dtype), x[:, :-k]], axis=1)
        x = x + shifted
        k *= 2
    return x


_CHUNK = 128


def _count_kernel(sd_ref, d_ref, r_ref, ty_ref, qy_ref, out_ref,
                  seen_ref, acc_ref, ct_ref, *, n_blocks, q, tb):
    b = pl.program_id(0)
    nch = tb // _CHUNK

    @pl.when(b == 0)
    def _init():
        seen_ref[...] = jnp.zeros((q, 1), jnp.float32)
        acc_ref[...] = jnp.zeros((q, 1), jnp.float32)
        ctw = tb // _CHUNK + 1
        chunk_id = jax.lax.broadcasted_iota(jnp.int32, (tb, ctw), 0) // _CHUNK
        col = jax.lax.broadcasted_iota(jnp.int32, (tb, ctw), 1)
        ct_ref[...] = (chunk_id < col).astype(jnp.float32)

    sd = sd_ref[...].astype(jnp.float32)
    dthr = d_ref[...] - 64.0
    match = (ty_ref[0] == qy_ref[...]).astype(jnp.float32)
    lt = (sd < dthr).astype(jnp.float32)
    eq = (sd == dthr).astype(jnp.float32)
    eqm = eq * match
    acc_ref[...] += jnp.sum(lt * match, axis=1, keepdims=True)

    ct = ct_ref[...]
    p_eq = jnp.dot(eq, ct, preferred_element_type=jnp.float32)
    p_em = jnp.dot(eqm, ct, preferred_element_type=jnp.float32)
    ec = p_eq[:, 1:nch + 1] - p_eq[:, :nch]
    em = p_em[:, 1:nch + 1] - p_em[:, :nch]
    s = (r_ref[...] - seen_ref[...]) - p_eq[:, :nch]
    ft = (s >= ec).astype(jnp.float32)
    pc = jnp.logical_and(s > 0.0, s < ec).astype(jnp.float32)
    acc_ref[...] += jnp.sum(ft * em, axis=1, keepdims=True)

    eqx = jnp.zeros((q, _CHUNK), jnp.float32)
    emx = jnp.zeros((q, _CHUNK), jnp.float32)
    for c in range(nch):
        w = pc[:, c:c + 1]
        eqx = eqx + w * eq[:, c * _CHUNK:(c + 1) * _CHUNK]
        emx = emx + w * eqm[:, c * _CHUNK:(c + 1) * _CHUNK]
    s_star = jnp.sum(pc * s, axis=1, keepdims=True)
    pre = _cumsum_lanes(eqx)
    acc_ref[...] += jnp.sum(emx * (pre <= s_star).astype(jnp.float32),
                            axis=1, keepdims=True)
    seen_ref[...] += p_eq[:, nch:nch + 1]

    @pl.when(b == n_blocks - 1)
    def _fin():
        out_ref[...] = (jnp.sum(acc_ref[...], keepdims=True)
                        / (q * float(_TOP_R)))


def kernel(train_f, train_y, test_f, test_y):
    q, k = test_f.shape
    n = train_f.shape[0]
    tb = _TB if n >= _TB else max(128, ((n + 127) // 128) * 128)
    nb = (n + tb - 1) // tb
    npad = nb * tb
    n_iters = 7

    train_f_pad = jnp.pad(train_f, ((0, npad - n), (0, 0)))
    ty = jnp.pad(train_y.astype(jnp.float32), (0, npad - n))
    ty3 = ty.reshape(nb, 1, tb)
    qy = test_y.astype(jnp.float32).reshape(q, 1)

    sdist, cnt0 = pl.pallas_call(
        functools.partial(_dist_kernel, n_train=n, tb=tb, n_blocks=nb),
        grid=(nb,),
        in_specs=[
            pl.BlockSpec((q, k), lambda b: (0, 0)),
            pl.BlockSpec((tb, k), lambda b: (b, 0)),
        ],
        out_specs=[
            pl.BlockSpec((q, tb), lambda b: (0, b)),
            pl.BlockSpec((q, 1), lambda b: (0, 0)),
        ],
        out_shape=[
            jax.ShapeDtypeStruct((q, npad), jnp.int8),
            jax.ShapeDtypeStruct((q, 1), jnp.float32),
        ],
        scratch_shapes=[pltpu.VMEM((q, 1), jnp.float32)],
    )(test_f, train_f_pad)

    dvals, rvals = pl.pallas_call(
        functools.partial(_search_kernel, n_blocks=nb, n_iters=n_iters,
                          q=q, tb=tb),
        grid=(n_iters, nb),
        in_specs=[pl.BlockSpec((q, tb), lambda it, b: (0, b)),
                  pl.BlockSpec((q, 1), lambda it, b: (0, 0))],
        out_specs=[
            pl.BlockSpec((q, 1), lambda it, b: (0, 0)),
            pl.BlockSpec((q, 1), lambda it, b: (0, 0)),
        ],
        out_shape=[
            jax.ShapeDtypeStruct((q, 1), jnp.float32),
            jax.ShapeDtypeStruct((q, 1), jnp.float32),
        ],
        scratch_shapes=[pltpu.VMEM((q, 1), jnp.float32) for _ in range(3)],
    )(sdist, cnt0)

    out = pl.pallas_call(
        functools.partial(_count_kernel, n_blocks=nb, q=q, tb=tb),
        grid=(nb,),
        in_specs=[
            pl.BlockSpec((q, tb), lambda b: (0, b)),
            pl.BlockSpec((q, 1), lambda b: (0, 0)),
            pl.BlockSpec((q, 1), lambda b: (0, 0)),
            pl.BlockSpec((1, 1, tb), lambda b: (b, 0, 0)),
            pl.BlockSpec((q, 1), lambda b: (0, 0)),
        ],
        out_specs=pl.BlockSpec((1, 1), lambda b: (0, 0)),
        out_shape=jax.ShapeDtypeStruct((1, 1), jnp.float32),
        scratch_shapes=(
            [pltpu.VMEM((q, 1), jnp.float32) for _ in range(2)]
            + [pltpu.VMEM((tb, tb // _CHUNK + 1), jnp.float32)]),
    )(sdist, dvals, rvals, ty3, qy)

    return out[0, 0]
